# baseline (speedup 1.0000x reference)
import functools

import jax
import jax.numpy as jnp
from jax.experimental import pallas as pl
from jax.experimental.pallas import tpu as pltpu

_VMEM_LIMIT = 64 * 1024 * 1024

_B0 = 8
_NUM_ACTIONS = 18


def _conv_stack_kernel(x_ref, t1_ref, b1_ref, t2_ref, b2_ref, t3_ref, b3_ref,
                       out_ref, st_ref, s1e_ref, s1o_ref, s2_ref, s3_ref):
    f32 = jnp.float32
    bf16 = jnp.bfloat16

    st_ref[:, :, 0:84, 0:84] = x_ref[...]
    st_ref[:, :, 84:, :] = jnp.zeros_like(st_ref[:, :, 84:, :])
    st_ref[:, :, :, 84:] = jnp.zeros_like(st_ref[:, :, :, 84:])

    for par, s1_ref in ((0, s1e_ref), (1, s1o_ref)):
        acc = None
        for c in range(4):
            for kh in range(8):
                xk = st_ref[:, c, pl.ds(kh + 4 * par, 16, 8), :]
                xk = xk.reshape(_B0 * 16, 128)
                r0 = (c * 8 + kh) * 128
                d = jnp.dot(xk.astype(bf16), t1_ref[pl.ds(r0, 128), :],
                            preferred_element_type=f32)
                acc = d if acc is None else acc + d
        h1 = jnp.maximum(acc + b1_ref[...], 0.0)
        s1_ref[:, 0:16, :] = h1.reshape(_B0, 16, 640)
        s1_ref[:, 16:, :] = jnp.zeros_like(s1_ref[:, 16:, :])

    acc = None
    for i in range(4):
        s1_ref = s1o_ref if i % 2 else s1e_ref
        xk = s1_ref[:, pl.ds(i // 2, 16), :].reshape(_B0 * 16, 640)
        d = jnp.dot(xk.astype(bf16), t2_ref[pl.ds(i * 640, 640), :],
                    preferred_element_type=f32)
        acc = d if acc is None else acc + d
    h2 = jnp.maximum(acc + b2_ref[...], 0.0)
    s2_ref[:, 0:16, :] = h2.reshape(_B0, 16, 640)
    s2_ref[:, 16:, :] = jnp.zeros_like(s2_ref[:, 16:, :])

    acc = None
    for i in range(3):
        xk = s2_ref[:, pl.ds(i, 16), :].reshape(_B0 * 16, 640)
        d = jnp.dot(xk.astype(bf16), t3_ref[pl.ds(i * 640, 640), :],
                    preferred_element_type=f32)
        acc = d if acc is None else acc + d
    s3_ref[...] = jnp.maximum(acc + b3_ref[...], 0.0).reshape(_B0, 16, 512)

    for oh in range(7):
        out_ref[:, pl.ds(oh * 448, 448)] = s3_ref[:, oh, 0:448]
    out_ref[:, 3136:] = jnp.zeros_like(out_ref[:, 3136:])


def _fc_stack_kernel(x_ref, w1_ref, b1_ref, w2_ref, b2_ref, w3_ref, b3_ref,
                     o_ref):
    f32 = jnp.float32
    h = jnp.dot(x_ref[...], w1_ref[...], preferred_element_type=f32) + b1_ref[...]
    h = jnp.maximum(h, 0.0)
    h = jnp.dot(h, w2_ref[...], preferred_element_type=f32) + b2_ref[...]
    o_ref[...] = (jnp.dot(h, w3_ref[...], preferred_element_type=f32)
                  + b3_ref[...])[:, :_NUM_ACTIONS]


def _build_toeplitz(c1_w, c1_b, c2_w, c2_b, c3_w, c3_b):
    bf16 = jnp.bfloat16
    w1r = c1_w[:256].reshape(8, 2, 4, 4, 128)[..., :32].astype(bf16)
    e1 = jnp.stack([jnp.eye(32, 20, k=0, dtype=bf16),
                    jnp.eye(32, 20, k=-1, dtype=bf16)])
    t1 = jnp.einsum("dwp,kdvco->ckwvpo", e1, w1r)
    t1 = t1.reshape(32 * 128, 640)
    b1t = jnp.tile(c1_b[:, :32], (1, 20))

    w2r = c2_w[:512].reshape(4, 4, 32, 128)[..., :64].astype(bf16)
    ow = jnp.arange(20)[:, None]
    ow2 = jnp.arange(10)[None, :]
    e2 = jnp.stack([(ow == 2 * ow2 + j).astype(bf16) for j in range(4)])
    t2 = jnp.einsum("jwp,ijco->iwcpo", e2, w2r)
    t2 = t2.reshape(4 * 640, 640)
    b2t = jnp.tile(c2_b[:, :64], (1, 10))

    w3r = c3_w[:576].reshape(3, 3, 64, 128)[..., :64].astype(bf16)
    e3 = jnp.stack([jnp.eye(10, 8, k=-j, dtype=bf16) for j in range(3)])
    t3 = jnp.einsum("jwp,ijco->iwcpo", e3, w3r)
    t3 = t3.reshape(3 * 640, 512)
    b3t = jnp.tile(c3_b[:, :64], (1, 8))
    return t1, b1t, t2, b2t, t3, b3t


def kernel(c1_w, c1_b, c2_w, c2_b, c3_w, c3_b,
           fc1_w, fc1_b, fc2_w, fc2_b, fc3_w, fc3_b, x):
    B = x.shape[0]
    assert B % _B0 == 0
    grid = B // _B0
    t1, b1t, t2, b2t, t3, b3t = _build_toeplitz(
        c1_w, c1_b, c2_w, c2_b, c3_w, c3_b)

    K1p = fc1_w.shape[0]
    assert K1p == 3200

    xf = pl.pallas_call(
        _conv_stack_kernel,
        out_shape=jax.ShapeDtypeStruct((B, K1p), jnp.float32),
        grid=(grid,),
        in_specs=[
            pl.BlockSpec((_B0, 4, 84, 84), lambda i: (i, 0, 0, 0)),
            pl.BlockSpec((32 * 128, 640), lambda i: (0, 0)),
            pl.BlockSpec((1, 640), lambda i: (0, 0)),
            pl.BlockSpec((4 * 640, 640), lambda i: (0, 0)),
            pl.BlockSpec((1, 640), lambda i: (0, 0)),
            pl.BlockSpec((3 * 640, 512), lambda i: (0, 0)),
            pl.BlockSpec((1, 512), lambda i: (0, 0)),
        ],
        out_specs=pl.BlockSpec((_B0, K1p), lambda i: (i, 0)),
        scratch_shapes=[
            pltpu.VMEM((_B0, 4, 136, 128), jnp.float32),
            pltpu.VMEM((_B0, 24, 640), jnp.float32),
            pltpu.VMEM((_B0, 24, 640), jnp.float32),
            pltpu.VMEM((_B0, 24, 640), jnp.float32),
            pltpu.VMEM((_B0, 16, 512), jnp.float32),
        ],
        compiler_params=pltpu.CompilerParams(
            dimension_semantics=("parallel",),
            vmem_limit_bytes=_VMEM_LIMIT,
        ),
    )(x, t1, b1t, t2, b2t, t3, b3t)

    out = pl.pallas_call(
        _fc_stack_kernel,
        out_shape=jax.ShapeDtypeStruct((B, _NUM_ACTIONS), jnp.float32),
        grid=(1,),
        in_specs=[
            pl.BlockSpec((B, K1p), lambda i: (0, 0)),
            pl.BlockSpec(fc1_w.shape, lambda i: (0, 0)),
            pl.BlockSpec((1, fc1_w.shape[1]), lambda i: (0, 0)),
            pl.BlockSpec(fc2_w.shape, lambda i: (0, 0)),
            pl.BlockSpec((1, fc2_w.shape[1]), lambda i: (0, 0)),
            pl.BlockSpec(fc3_w.shape, lambda i: (0, 0)),
            pl.BlockSpec((1, fc3_w.shape[1]), lambda i: (0, 0)),
        ],
        out_specs=pl.BlockSpec((B, _NUM_ACTIONS), lambda i: (0, 0)),
        compiler_params=pltpu.CompilerParams(
            dimension_semantics=("arbitrary",),
            vmem_limit_bytes=_VMEM_LIMIT,
        ),
    )(xf, fc1_w, fc1_b, fc2_w, fc2_b, fc3_w, fc3_b)
    return out


# speedup vs baseline: 2.1429x; 2.1429x over previous
import functools

import jax
import jax.numpy as jnp
from jax.experimental import pallas as pl
from jax.experimental.pallas import tpu as pltpu

_VMEM_LIMIT = 64 * 1024 * 1024

_B0 = 8
_NUM_ACTIONS = 18


def _conv_stack_kernel(xp_ref, w1_ref, b1_ref, w2_ref, b2_ref, w3_ref, b3_ref,
                       out_ref, s0_ref, s1_ref, s2_ref, s3_ref):
    f32 = jnp.float32

    s0_ref[:, :, 0:21, :] = xp_ref[...]
    s0_ref[:, :, 21:, :] = jnp.zeros_like(s0_ref[:, :, 21:, :])

    s1_ref[:, :, 24:, :] = jnp.zeros_like(s1_ref[:, :, 24:, :])
    for bs in range(0, _B0, 2):
        taps = []
        for di in (0, 1):
            for dj in (0, 1):
                taps.append(s0_ref[bs:bs + 2, di:di + 20, pl.ds(dj, 24), :])
        x = jnp.concatenate(taps, axis=-1).reshape(2 * 20 * 24, 256)
        h = jnp.dot(x, w1_ref[...], preferred_element_type=f32) + b1_ref[...]
        h = jnp.maximum(h, 0.0)
        s1_ref[bs:bs + 2, :, 0:24, :] = h.reshape(2, 20, 24, 128)[..., 0:32]

    for bs in range(0, _B0, 4):
        taps = []
        for i in range(4):
            for j in range(4):
                taps.append(s1_ref[bs:bs + 4, pl.ds(i, 9, 2), pl.ds(j, 16, 2), :])
        x = jnp.concatenate(taps, axis=-1).reshape(4 * 9 * 16, 512)
        h = jnp.dot(x, w2_ref[...], preferred_element_type=f32) + b2_ref[...]
        h = jnp.maximum(h, 0.0)
        s2_ref[bs:bs + 4] = h.reshape(4, 9, 16, 128)[..., 0:64]

    taps = []
    for i in range(3):
        for j in range(3):
            taps.append(s2_ref[:, i:i + 7, pl.ds(j, 8), :])
    x = jnp.concatenate(taps, axis=-1).reshape(_B0 * 7 * 8, 576)
    h = jnp.dot(x, w3_ref[...], preferred_element_type=f32) + b3_ref[...]
    h = jnp.maximum(h, 0.0)
    s3_ref[...] = h.reshape(_B0, 7, 8, 128)[..., 0:64]

    for oh in range(7):
        for ow in range(7):
            p = oh * 7 + ow
            out_ref[:, pl.ds(p * 64, 64)] = s3_ref[:, oh, ow, :]
    out_ref[:, 3136:] = jnp.zeros_like(out_ref[:, 3136:])


def _fc_stack_kernel(x_ref, w1_ref, b1_ref, w2_ref, b2_ref, w3_ref, b3_ref,
                     o_ref):
    f32 = jnp.float32
    h = jnp.dot(x_ref[...], w1_ref[...], preferred_element_type=f32) + b1_ref[...]
    h = jnp.maximum(h, 0.0)
    h = jnp.dot(h, w2_ref[...], preferred_element_type=f32) + b2_ref[...]
    o_ref[...] = (jnp.dot(h, w3_ref[...], preferred_element_type=f32)
                  + b3_ref[...])[:, :_NUM_ACTIONS]


def kernel(c1_w, c1_b, c2_w, c2_b, c3_w, c3_b,
           fc1_w, fc1_b, fc2_w, fc2_b, fc3_w, fc3_b, x):
    B = x.shape[0]
    assert B % _B0 == 0
    grid = B // _B0
    bf16 = jnp.bfloat16

    xp = x.astype(bf16).reshape(B, 4, 21, 4, 21, 4)
    xp = xp.transpose(0, 2, 4, 1, 3, 5).reshape(B, 21, 21, 64)

    w1 = c1_w[:256].reshape(2, 4, 2, 4, 4, 128)
    w1 = w1.transpose(0, 2, 4, 1, 3, 5).reshape(256, 128).astype(bf16)
    w3 = c3_w[:576]

    K1p = fc1_w.shape[0]
    assert K1p == 3200

    xf = pl.pallas_call(
        _conv_stack_kernel,
        out_shape=jax.ShapeDtypeStruct((B, K1p), jnp.float32),
        grid=(grid,),
        in_specs=[
            pl.BlockSpec((_B0, 21, 21, 64), lambda i: (i, 0, 0, 0)),
            pl.BlockSpec((256, 128), lambda i: (0, 0)),
            pl.BlockSpec((1, 128), lambda i: (0, 0)),
            pl.BlockSpec((512, 128), lambda i: (0, 0)),
            pl.BlockSpec((1, 128), lambda i: (0, 0)),
            pl.BlockSpec((576, 128), lambda i: (0, 0)),
            pl.BlockSpec((1, 128), lambda i: (0, 0)),
        ],
        out_specs=pl.BlockSpec((_B0, K1p), lambda i: (i, 0)),
        scratch_shapes=[
            pltpu.VMEM((_B0, 21, 32, 64), bf16),
            pltpu.VMEM((_B0, 20, 40, 32), jnp.float32),
            pltpu.VMEM((_B0, 9, 16, 64), jnp.float32),
            pltpu.VMEM((_B0, 7, 8, 64), jnp.float32),
        ],
        compiler_params=pltpu.CompilerParams(
            dimension_semantics=("parallel",),
            vmem_limit_bytes=_VMEM_LIMIT,
        ),
    )(xp, w1, c1_b, c2_w, c2_b, w3, c3_b)

    out = pl.pallas_call(
        _fc_stack_kernel,
        out_shape=jax.ShapeDtypeStruct((B, _NUM_ACTIONS), jnp.float32),
        grid=(1,),
        in_specs=[
            pl.BlockSpec((B, K1p), lambda i: (0, 0)),
            pl.BlockSpec(fc1_w.shape, lambda i: (0, 0)),
            pl.BlockSpec((1, fc1_w.shape[1]), lambda i: (0, 0)),
            pl.BlockSpec(fc2_w.shape, lambda i: (0, 0)),
            pl.BlockSpec((1, fc2_w.shape[1]), lambda i: (0, 0)),
            pl.BlockSpec(fc3_w.shape, lambda i: (0, 0)),
            pl.BlockSpec((1, fc3_w.shape[1]), lambda i: (0, 0)),
        ],
        out_specs=pl.BlockSpec((B, _NUM_ACTIONS), lambda i: (0, 0)),
        compiler_params=pltpu.CompilerParams(
            dimension_semantics=("arbitrary",),
            vmem_limit_bytes=_VMEM_LIMIT,
        ),
    )(xf, fc1_w, fc1_b, fc2_w, fc2_b, fc3_w, fc3_b)
    return out
